# baseline (speedup 1.0000x reference)
import functools

import jax
import jax.numpy as jnp
from jax.experimental import pallas as pl


def _pick_row_block(n):
    for cand in (200, 80, 40, 16, 8):
        if n % cand == 0:
            return cand
    return n


def _deg_cast_kernel(adj_ref, deg_ref, adjb_ref):
    m = adj_ref.shape[0]
    a = adj_ref[...]
    s = jnp.sum(a, axis=1) + 1.0
    deg_ref[...] = s.reshape(1, 1, m)
    adjb_ref[...] = a.astype(jnp.bfloat16)


def _scale_matmul_kernel(x_ref, w_ref, deg_ref, out_ref):
    deg = deg_ref[...]
    dinv = jnp.where(deg > 0, jax.lax.rsqrt(deg), 0.0)
    out_ref[...] = (
        dinv * jnp.dot(x_ref[...], w_ref[...], preferred_element_type=jnp.float32)
    ).astype(jnp.bfloat16)


def _layer1_kernel(adj_ref, y_ref, yself_ref, deg_ref, b_ref, w2_ref, out_ref):
    deg = deg_ref[...]
    dinv = jnp.where(deg > 0, jax.lax.rsqrt(deg), 0.0)
    acc = jnp.dot(adj_ref[...], y_ref[...], preferred_element_type=jnp.float32)
    x1 = dinv * (acc + yself_ref[...].astype(jnp.float32)) + b_ref[...]
    x1 = jnp.maximum(x1, 0.0)
    out_ref[...] = (
        dinv * jnp.dot(x1, w2_ref[...], preferred_element_type=jnp.float32)
    ).astype(jnp.bfloat16)


def _layer2_kernel(adj_ref, y_ref, yself_ref, deg_ref, b_ref, out_ref):
    deg = deg_ref[...]
    dinv = jnp.where(deg > 0, jax.lax.rsqrt(deg), 0.0)
    acc = jnp.dot(adj_ref[...], y_ref[...], preferred_element_type=jnp.float32)
    out_ref[...] = dinv * (acc + yself_ref[...].astype(jnp.float32)) + b_ref[...]


@jax.jit
def kernel(feature, adj, W1, b1, W2, b2):
    n, d = feature.shape
    h1 = W1.shape[1]
    h2 = W2.shape[1]
    mb = _pick_row_block(n)
    nmb = n // mb

    deg3, adjb = pl.pallas_call(
        _deg_cast_kernel,
        grid=(nmb,),
        in_specs=[pl.BlockSpec((mb, n), lambda i: (i, 0))],
        out_specs=[
            pl.BlockSpec((1, 1, mb), lambda i: (i, 0, 0)),
            pl.BlockSpec((mb, n), lambda i: (i, 0)),
        ],
        out_shape=[
            jax.ShapeDtypeStruct((nmb, 1, mb), jnp.float32),
            jax.ShapeDtypeStruct((n, n), jnp.bfloat16),
        ],
    )(adj)
    deg = deg3.reshape(n, 1)

    y1 = pl.pallas_call(
        _scale_matmul_kernel,
        out_shape=jax.ShapeDtypeStruct((n, h1), jnp.bfloat16),
    )(feature, W1, deg)

    b1r = b1.reshape(1, h1)
    b2r = b2.reshape(1, h2)

    y2 = pl.pallas_call(
        _layer1_kernel,
        grid=(nmb,),
        in_specs=[
            pl.BlockSpec((mb, n), lambda i: (i, 0)),
            pl.BlockSpec((n, h1), lambda i: (0, 0)),
            pl.BlockSpec((mb, h1), lambda i: (i, 0)),
            pl.BlockSpec((mb, 1), lambda i: (i, 0)),
            pl.BlockSpec((1, h1), lambda i: (0, 0)),
            pl.BlockSpec((h1, h2), lambda i: (0, 0)),
        ],
        out_specs=pl.BlockSpec((mb, h2), lambda i: (i, 0)),
        out_shape=jax.ShapeDtypeStruct((n, h2), jnp.bfloat16),
    )(adjb, y1, y1, deg, b1r, W2)

    x2 = pl.pallas_call(
        _layer2_kernel,
        grid=(nmb,),
        in_specs=[
            pl.BlockSpec((mb, n), lambda i: (i, 0)),
            pl.BlockSpec((n, h2), lambda i: (0, 0)),
            pl.BlockSpec((mb, h2), lambda i: (i, 0)),
            pl.BlockSpec((mb, 1), lambda i: (i, 0)),
            pl.BlockSpec((1, h2), lambda i: (0, 0)),
        ],
        out_specs=pl.BlockSpec((mb, h2), lambda i: (i, 0)),
        out_shape=jax.ShapeDtypeStruct((n, h2), jnp.float32),
    )(adjb, y2, y2, deg, b2r)

    return x2


# speedup vs baseline: 1.0981x; 1.0981x over previous
import functools

import jax
import jax.numpy as jnp
from jax.experimental import pallas as pl


def _pick_row_block(n, pref):
    for cand in pref:
        if n % cand == 0:
            return cand
    return n


def _deg_cast_kernel(adj_ref, deg_ref, adjb_ref):
    m = adj_ref.shape[0]
    a = adj_ref[...]
    s = jnp.sum(a, axis=1) + 1.0
    deg_ref[...] = s.reshape(1, 1, m)
    adjb_ref[...] = a.astype(jnp.bfloat16)


def _scale_matmul_kernel(x_ref, w_ref, deg_ref, out_ref):
    deg = deg_ref[...]
    dinv = jnp.where(deg > 0, jax.lax.rsqrt(deg), 0.0)
    out_ref[...] = (
        dinv * jnp.dot(x_ref[...], w_ref[...], preferred_element_type=jnp.float32)
    ).astype(jnp.bfloat16)


def _layer1_kernel(adj_ref, y_ref, yself_ref, deg_ref, b_ref, w2_ref, out_ref):
    deg = deg_ref[...]
    dinv = jnp.where(deg > 0, jax.lax.rsqrt(deg), 0.0)
    acc = jnp.dot(adj_ref[...], y_ref[...], preferred_element_type=jnp.float32)
    x1 = dinv * (acc + yself_ref[...].astype(jnp.float32)) + b_ref[...]
    x1 = jnp.maximum(x1, 0.0)
    out_ref[...] = (
        dinv * jnp.dot(x1, w2_ref[...], preferred_element_type=jnp.float32)
    ).astype(jnp.bfloat16)


def _layer2_kernel(adj_ref, y_ref, yself_ref, deg_ref, b_ref, out_ref):
    deg = deg_ref[...]
    dinv = jnp.where(deg > 0, jax.lax.rsqrt(deg), 0.0)
    acc = jnp.dot(adj_ref[...], y_ref[...], preferred_element_type=jnp.float32)
    out_ref[...] = dinv * (acc + yself_ref[...].astype(jnp.float32)) + b_ref[...]


@jax.jit
def kernel(feature, adj, W1, b1, W2, b2):
    n, d = feature.shape
    h1 = W1.shape[1]
    h2 = W2.shape[1]
    mb = _pick_row_block(n, (400, 200, 80, 40, 16, 8))
    nmb = n // mb
    lb = _pick_row_block(n, (1000, 400, 200, 80, 40, 16, 8))
    nlb = n // lb

    deg3, adjb = pl.pallas_call(
        _deg_cast_kernel,
        grid=(nmb,),
        in_specs=[pl.BlockSpec((mb, n), lambda i: (i, 0))],
        out_specs=[
            pl.BlockSpec((1, 1, mb), lambda i: (i, 0, 0)),
            pl.BlockSpec((mb, n), lambda i: (i, 0)),
        ],
        out_shape=[
            jax.ShapeDtypeStruct((nmb, 1, mb), jnp.float32),
            jax.ShapeDtypeStruct((n, n), jnp.bfloat16),
        ],
    )(adj)
    deg = deg3.reshape(n, 1)

    y1 = pl.pallas_call(
        _scale_matmul_kernel,
        out_shape=jax.ShapeDtypeStruct((n, h1), jnp.bfloat16),
    )(feature, W1, deg)

    b1r = b1.reshape(1, h1)
    b2r = b2.reshape(1, h2)

    y2 = pl.pallas_call(
        _layer1_kernel,
        grid=(nlb,),
        in_specs=[
            pl.BlockSpec((lb, n), lambda i: (i, 0)),
            pl.BlockSpec((n, h1), lambda i: (0, 0)),
            pl.BlockSpec((lb, h1), lambda i: (i, 0)),
            pl.BlockSpec((lb, 1), lambda i: (i, 0)),
            pl.BlockSpec((1, h1), lambda i: (0, 0)),
            pl.BlockSpec((h1, h2), lambda i: (0, 0)),
        ],
        out_specs=pl.BlockSpec((lb, h2), lambda i: (i, 0)),
        out_shape=jax.ShapeDtypeStruct((n, h2), jnp.bfloat16),
    )(adjb, y1, y1, deg, b1r, W2)

    x2 = pl.pallas_call(
        _layer2_kernel,
        grid=(nlb,),
        in_specs=[
            pl.BlockSpec((lb, n), lambda i: (i, 0)),
            pl.BlockSpec((n, h2), lambda i: (0, 0)),
            pl.BlockSpec((lb, h2), lambda i: (i, 0)),
            pl.BlockSpec((lb, 1), lambda i: (i, 0)),
            pl.BlockSpec((1, h2), lambda i: (0, 0)),
        ],
        out_specs=pl.BlockSpec((lb, h2), lambda i: (i, 0)),
        out_shape=jax.ShapeDtypeStruct((n, h2), jnp.float32),
    )(adjb, y2, y2, deg, b2r)

    return x2


# speedup vs baseline: 1.1302x; 1.0292x over previous
import jax
import jax.numpy as jnp
from jax.experimental import pallas as pl
from jax.experimental.pallas import tpu as pltpu


def _pick_row_block(n, pref):
    for cand in pref:
        if n % cand == 0:
            return cand
    return n


def _dinv(deg):
    return jnp.where(deg > 0, jax.lax.rsqrt(deg), 0.0)


def _pass1_kernel(adj_ref, x_ref, w1_ref, deg_ref, adjb_ref, z_ref):
    m = adj_ref.shape[0]
    a = adj_ref[...]
    deg_ref[...] = (jnp.sum(a, axis=1) + 1.0).reshape(m, 1)
    adjb_ref[...] = a.astype(jnp.bfloat16)
    z_ref[...] = jnp.dot(x_ref[...], w1_ref[...], preferred_element_type=jnp.float32)


def _layer1_kernel(adj_ref, z_ref, degf_ref, deg_ref, b_ref, w2_ref, out_ref, y_scr):
    i = pl.program_id(0)
    m = adj_ref.shape[0]

    @pl.when(i == 0)
    def _():
        y_scr[...] = (_dinv(degf_ref[...]) * z_ref[...]).astype(jnp.bfloat16)

    dinv = _dinv(deg_ref[...])
    acc = jnp.dot(adj_ref[...], y_scr[...], preferred_element_type=jnp.float32)
    yself = y_scr[pl.ds(i * m, m), :].astype(jnp.float32)
    x1 = dinv * (acc + yself) + b_ref[...]
    x1 = jnp.maximum(x1, 0.0)
    out_ref[...] = (
        dinv * jnp.dot(x1, w2_ref[...], preferred_element_type=jnp.float32)
    ).astype(jnp.bfloat16)


def _layer2_kernel(adj_ref, y_ref, yself_ref, deg_ref, b_ref, out_ref):
    dinv = _dinv(deg_ref[...])
    acc = jnp.dot(adj_ref[...], y_ref[...], preferred_element_type=jnp.float32)
    out_ref[...] = dinv * (acc + yself_ref[...].astype(jnp.float32)) + b_ref[...]


@jax.jit
def kernel(feature, adj, W1, b1, W2, b2):
    n, d = feature.shape
    h1 = W1.shape[1]
    h2 = W2.shape[1]
    mb = _pick_row_block(n, (400, 200, 80, 40, 16, 8))
    nmb = n // mb
    lb = _pick_row_block(n, (1000, 400, 200, 80, 40, 16, 8))
    nlb = n // lb

    deg, adjb, z = pl.pallas_call(
        _pass1_kernel,
        grid=(nmb,),
        in_specs=[
            pl.BlockSpec((mb, n), lambda i: (i, 0)),
            pl.BlockSpec((mb, d), lambda i: (i, 0)),
            pl.BlockSpec((d, h1), lambda i: (0, 0)),
        ],
        out_specs=[
            pl.BlockSpec((mb, 1), lambda i: (i, 0)),
            pl.BlockSpec((mb, n), lambda i: (i, 0)),
            pl.BlockSpec((mb, h1), lambda i: (i, 0)),
        ],
        out_shape=[
            jax.ShapeDtypeStruct((n, 1), jnp.float32),
            jax.ShapeDtypeStruct((n, n), jnp.bfloat16),
            jax.ShapeDtypeStruct((n, h1), jnp.float32),
        ],
    )(adj, feature, W1)

    b1r = b1.reshape(1, h1)
    b2r = b2.reshape(1, h2)

    y2 = pl.pallas_call(
        _layer1_kernel,
        grid=(nlb,),
        in_specs=[
            pl.BlockSpec((lb, n), lambda i: (i, 0)),
            pl.BlockSpec((n, h1), lambda i: (0, 0)),
            pl.BlockSpec((n, 1), lambda i: (0, 0)),
            pl.BlockSpec((lb, 1), lambda i: (i, 0)),
            pl.BlockSpec((1, h1), lambda i: (0, 0)),
            pl.BlockSpec((h1, h2), lambda i: (0, 0)),
        ],
        out_specs=pl.BlockSpec((lb, h2), lambda i: (i, 0)),
        out_shape=jax.ShapeDtypeStruct((n, h2), jnp.bfloat16),
        scratch_shapes=[pltpu.VMEM((n, h1), jnp.bfloat16)],
    )(adjb, z, deg, deg, b1r, W2)

    x2 = pl.pallas_call(
        _layer2_kernel,
        grid=(nlb,),
        in_specs=[
            pl.BlockSpec((lb, n), lambda i: (i, 0)),
            pl.BlockSpec((n, h2), lambda i: (0, 0)),
            pl.BlockSpec((lb, h2), lambda i: (i, 0)),
            pl.BlockSpec((lb, 1), lambda i: (i, 0)),
            pl.BlockSpec((1, h2), lambda i: (0, 0)),
        ],
        out_specs=pl.BlockSpec((lb, h2), lambda i: (i, 0)),
        out_shape=jax.ShapeDtypeStruct((n, h2), jnp.float32),
    )(adjb, y2, y2, deg, b2r)

    return x2
